# baseline (speedup 1.0000x reference)
import functools

import jax
import jax.numpy as jnp
from jax import lax
from jax.experimental import pallas as pl
from jax.experimental.pallas import tpu as pltpu
from jax.experimental.pallas import tpu_sc as plsc

_VOCAB = 49408
_SEQ = 77
_DIM = 768
_BATCH = 1024
_ROWS = _BATCH * _SEQ
_NW = 32
_RPW = _ROWS // _NW
_CHUNK = 16
_NCH = _RPW // _CHUNK
_LANES = 16
_CB = _DIM // _LANES


def _embed_body(ids_hbm, emb_hbm, pos_hbm, out_hbm, idx_v, pos_v, a_v, sem):
    wid = lax.axis_index("s") * 2 + lax.axis_index("c")
    row_base = wid * _RPW
    pltpu.sync_copy(ids_hbm.at[wid], idx_v)
    pltpu.sync_copy(pos_hbm, pos_v)

    def chunk_body(c, carry):
        pltpu.async_copy(emb_hbm.at[idx_v.at[c]], a_v, sem).wait()
        cbase = c * _CHUNK

        def row_body(r, carry2):
            prow = lax.rem(cbase + r, _SEQ)
            for j in range(_CB):
                sl = pl.ds(j * _LANES, _LANES)
                a_v[r, sl] = a_v[r, sl] + pos_v[prow, sl]
            return carry2

        lax.fori_loop(0, _CHUNK, row_body, 0, unroll=False)
        pltpu.sync_copy(a_v, out_hbm.at[pl.ds(row_base + c * _CHUNK, _CHUNK)])
        return carry

    lax.fori_loop(0, _NCH, chunk_body, 0, unroll=False)


@jax.jit
def _embed(ids2d, emb, pos):
    mesh = plsc.VectorSubcoreMesh(core_axis_name="c", subcore_axis_name="s")
    kern = functools.partial(
        pl.kernel,
        mesh=mesh,
        out_type=jax.ShapeDtypeStruct((_ROWS, _DIM), jnp.float32),
        scratch_types=[
            pltpu.VMEM((_NCH, _CHUNK), jnp.int32),
            pltpu.VMEM((_SEQ, _DIM), jnp.float32),
            pltpu.VMEM((_CHUNK, _DIM), jnp.float32),
            pltpu.SemaphoreType.DMA,
        ],
    )(_embed_body)
    return kern(ids2d, emb, pos)


def kernel(input_ids, embed_w, pos_embed_w):
    ids2d = input_ids.astype(jnp.int32).reshape(_NW, _NCH, _CHUNK)
    out = _embed(ids2d, embed_w, pos_embed_w)
    return out.reshape(_BATCH, _SEQ, _DIM)


# speedup vs baseline: 1.2956x; 1.2956x over previous
import functools

import jax
import jax.numpy as jnp
from jax import lax
from jax.experimental import pallas as pl
from jax.experimental.pallas import tpu as pltpu
from jax.experimental.pallas import tpu_sc as plsc

_VOCAB = 49408
_SEQ = 77
_DIM = 768
_BATCH = 1024
_ROWS = _BATCH * _SEQ
_NW = 32
_RPW = _ROWS // _NW
_CHUNK = 8
_NCH = _RPW // _CHUNK
_NBUF = 4
_LANES = 16
_CB = _DIM // _LANES


def _embed_body(ids_hbm, emb_hbm, pos_hbm, out_hbm,
                idx_v, pos_v, a0, a1, a2, a3,
                g0, g1, g2, g3, o0, o1, o2, o3):
    bufs = (a0, a1, a2, a3)
    gsems = (g0, g1, g2, g3)
    osems = (o0, o1, o2, o3)
    wid = lax.axis_index("s") * 2 + lax.axis_index("c")
    row_base = wid * _RPW
    pltpu.sync_copy(ids_hbm.at[wid], idx_v)
    pltpu.async_copy(emb_hbm.at[idx_v.at[0]], a0, g0)
    pltpu.async_copy(emb_hbm.at[idx_v.at[1]], a1, g1)
    pltpu.sync_copy(pos_hbm, pos_v)

    def outer(i, carry):
        for b in range(_NBUF):
            c = i * _NBUF + b
            bn = (b + 2) % _NBUF
            a_v = bufs[b]
            pltpu.make_async_copy(emb_hbm.at[idx_v.at[c]], a_v, gsems[b]).wait()

            def row_body(r, carry2):
                prow = lax.rem(c * _CHUNK + r, _SEQ)
                for j in range(_CB):
                    sl = pl.ds(j * _LANES, _LANES)
                    a_v[r, sl] = a_v[r, sl] + pos_v[prow, sl]
                return carry2

            lax.fori_loop(0, _CHUNK, row_body, 0, unroll=False)
            pltpu.async_copy(
                a_v, out_hbm.at[pl.ds(row_base + c * _CHUNK, _CHUNK)], osems[b])

            @pl.when(c + 2 < _NCH)
            def _issue_next():
                @pl.when(c >= 2)
                def _drain_prev():
                    pltpu.make_async_copy(
                        bufs[bn],
                        out_hbm.at[pl.ds(row_base + (c - 2) * _CHUNK, _CHUNK)],
                        osems[bn]).wait()

                pltpu.async_copy(emb_hbm.at[idx_v.at[c + 2]], bufs[bn], gsems[bn])

        return carry

    lax.fori_loop(0, _NCH // _NBUF, outer, 0, unroll=False)
    for b in range(_NBUF):
        c = _NCH - _NBUF + b
        pltpu.make_async_copy(
            bufs[c % _NBUF],
            out_hbm.at[pl.ds(row_base + c * _CHUNK, _CHUNK)],
            osems[c % _NBUF]).wait()


@jax.jit
def _embed(ids3d, emb, pos):
    mesh = plsc.VectorSubcoreMesh(core_axis_name="c", subcore_axis_name="s")
    kern = functools.partial(
        pl.kernel,
        mesh=mesh,
        out_type=jax.ShapeDtypeStruct((_ROWS, _DIM), jnp.float32),
        scratch_types=[
            pltpu.VMEM((_NCH, _CHUNK), jnp.int32),
            pltpu.VMEM((_SEQ, _DIM), jnp.float32),
        ] + [pltpu.VMEM((_CHUNK, _DIM), jnp.float32)] * _NBUF
          + [pltpu.SemaphoreType.DMA] * (2 * _NBUF),
    )(_embed_body)
    return kern(ids3d, emb, pos)


def kernel(input_ids, embed_w, pos_embed_w):
    ids3d = input_ids.astype(jnp.int32).reshape(_NW, _NCH, _CHUNK)
    out = _embed(ids3d, embed_w, pos_embed_w)
    return out.reshape(_BATCH, _SEQ, _DIM)


# speedup vs baseline: 1.8333x; 1.4150x over previous
import functools

import jax
import jax.numpy as jnp
from jax import lax
from jax.experimental import pallas as pl
from jax.experimental.pallas import tpu as pltpu
from jax.experimental.pallas import tpu_sc as plsc

_VOCAB = 49408
_SEQ = 77
_SEQP = 80
_MAIN = 72
_TAIL = _SEQ - _MAIN
_DIM = 768
_BATCH = 1024
_NW = 32
_SPW = _BATCH // _NW
_LANES = 16
_CB = _DIM // _LANES


def _embed_body(ids_hbm, emb_hbm, pos_hbm, out_hbm,
                idx_v, pos_v, a_v, t_v, gsem):
    wid = lax.axis_index("s") * 2 + lax.axis_index("c")
    batch_base = wid * _SPW

    def issue_gathers():
        pltpu.async_copy(emb_hbm.at[idx_v.at[0, pl.ds(0, _MAIN)]],
                         a_v.at[pl.ds(0, _MAIN)], gsem)
        pltpu.async_copy(emb_hbm.at[idx_v.at[0, pl.ds(_MAIN, 8)]], t_v, gsem)

    def wait_gathers():
        pltpu.make_async_copy(emb_hbm.at[idx_v.at[0, pl.ds(0, _MAIN)]],
                              a_v.at[pl.ds(0, _MAIN)], gsem).wait()
        pltpu.make_async_copy(emb_hbm.at[idx_v.at[0, pl.ds(_MAIN, 8)]],
                              t_v, gsem).wait()

    pltpu.sync_copy(ids_hbm.at[wid, 0], idx_v)
    issue_gathers()
    pltpu.sync_copy(pos_hbm, pos_v)

    def seq_body(s, carry):
        wait_gathers()
        @pl.when(s + 1 < _SPW)
        def _stage():
            pltpu.sync_copy(ids_hbm.at[wid, s + 1], idx_v)

        def row(r, carry2):
            for j in range(_CB):
                sl = pl.ds(j * _LANES, _LANES)
                a_v[r, sl] = a_v[r, sl] + pos_v[r, sl]
            return carry2

        lax.fori_loop(0, _MAIN, row, 0, unroll=False)

        def tail(k, carry2):
            for j in range(_CB):
                sl = pl.ds(j * _LANES, _LANES)
                a_v[_MAIN + k, sl] = t_v[k, sl] + pos_v[_MAIN + k, sl]
            return carry2

        lax.fori_loop(0, _TAIL, tail, 0, unroll=False)
        pltpu.sync_copy(a_v, out_hbm.at[batch_base + s])

        @pl.when(s + 1 < _SPW)
        def _next():
            issue_gathers()

        return carry

    lax.fori_loop(0, _SPW, seq_body, 0, unroll=False)


@jax.jit
def _embed(ids3d, emb, pos):
    mesh = plsc.VectorSubcoreMesh(core_axis_name="c", subcore_axis_name="s")
    kern = functools.partial(
        pl.kernel,
        mesh=mesh,
        out_type=jax.ShapeDtypeStruct((_BATCH, _SEQ, _DIM), jnp.float32),
        scratch_types=[
            pltpu.VMEM((1, _SEQP), jnp.int32),
            pltpu.VMEM((_SEQ, _DIM), jnp.float32),
            pltpu.VMEM((_SEQ, _DIM), jnp.float32),
            pltpu.VMEM((8, _DIM), jnp.float32),
            pltpu.SemaphoreType.DMA,
        ],
    )(_embed_body)
    return kern(ids3d, emb, pos)


def kernel(input_ids, embed_w, pos_embed_w):
    ids = input_ids.astype(jnp.int32)
    ids_pad = jnp.concatenate(
        [ids, jnp.zeros((_BATCH, _SEQP - _SEQ), jnp.int32)], axis=1)
    ids4d = ids_pad.reshape(_NW, _SPW, 1, _SEQP)
    return _embed(ids4d, embed_w, pos_embed_w)
